# baseline (speedup 1.0000x reference)
import functools

import jax
import jax.numpy as jnp
from jax import lax
from jax.experimental import pallas as pl
from jax.experimental.pallas import tpu as pltpu
from jax.experimental.pallas import tpu_sc as plsc

B, L, D = 16384, 50, 64
BL = B * L

CHUNK = 128
_NW = 32
N_PER_W = BL // _NW
N_CHUNKS = N_PER_W // CHUNK
K_INFLIGHT = 8
N_GROUPS = N_CHUNKS // K_INFLIGHT


def _make_sc_gather():
    info = plsc.get_sparse_core_info()
    nc, ns = info.num_cores, info.num_subcores
    assert nc * ns == _NW
    mesh = plsc.VectorSubcoreMesh(core_axis_name="c", subcore_axis_name="s")

    @functools.partial(
        pl.kernel,
        mesh=mesh,
        out_type=jax.ShapeDtypeStruct((_NW, N_CHUNKS, CHUNK), jnp.float32),
        scratch_types=[
            pltpu.VMEM((N_CHUNKS, CHUNK), jnp.int32),
            pltpu.VMEM((N_CHUNKS, CHUNK), jnp.float32),
            pltpu.SemaphoreType.DMA,
        ],
    )
    def gather_k(table_hbm, idx_hbm, out_hbm, idx_v, rows_v, sem):
        wid = lax.axis_index("s") * nc + lax.axis_index("c")
        pltpu.sync_copy(idx_hbm.at[wid], idx_v)

        def group(g, _):
            base = g * K_INFLIGHT
            descs = []
            for k in range(K_INFLIGHT):
                descs.append(
                    pltpu.async_copy(
                        table_hbm.at[idx_v.at[base + k]], rows_v.at[base + k], sem
                    )
                )
            for d in descs:
                d.wait()
            return 0

        lax.fori_loop(0, N_GROUPS, group, 0)
        pltpu.sync_copy(rows_v, out_hbm.at[wid])

    return gather_k


_sc_gather = _make_sc_gather()


BBLK = 1024


def _tc_body(x_ref, w_ref, bvec_ref, bias_ref, p_ref):
    x = x_ref[...].reshape(L, D, BBLK)
    w3 = w_ref[...].reshape(1, D, 1)
    s = jnp.sum(x * w3, axis=1)
    c = jnp.sum(bias_ref[...] * w_ref[...]) + bvec_ref[0, 0]
    p_ref[...] = jax.nn.sigmoid(s + c)


MBLK = 4096


def _fin_body(g_ref, p_ref, eff_ref, pout_ref):
    p = p_ref[...]
    eff = g_ref[...] * p
    eff_ref[...] = eff.T
    pout_ref[...] = p.T


def kernel(variantxgene_embedding, variantxgene_ixs, W, b, variantxgene_effect, embedding_bias):
    idx_t = (
        variantxgene_ixs.astype(jnp.int32)
        .T.reshape(_NW, N_CHUNKS, CHUNK)
    )
    gathered = _sc_gather(variantxgene_effect, idx_t).reshape(L, B)

    x2 = variantxgene_embedding.transpose(1, 2, 0).reshape(L * D, B)

    w_col = W.reshape(D, 1)
    bias_col = embedding_bias.reshape(D, 1)
    b2 = b.reshape(1, 1)

    grid = (B // BBLK,)
    p_t = pl.pallas_call(
        _tc_body,
        grid=grid,
        in_specs=[
            pl.BlockSpec((L * D, BBLK), lambda i: (0, i)),
            pl.BlockSpec((D, 1), lambda i: (0, 0)),
            pl.BlockSpec((1, 1), lambda i: (0, 0)),
            pl.BlockSpec((D, 1), lambda i: (0, 0)),
        ],
        out_specs=pl.BlockSpec((L, BBLK), lambda i: (0, i)),
        out_shape=jax.ShapeDtypeStruct((L, B), jnp.float32),
    )(x2, w_col, b2, bias_col)

    effect, prioritization = pl.pallas_call(
        _fin_body,
        grid=(B // MBLK,),
        in_specs=[
            pl.BlockSpec((L, MBLK), lambda i: (0, i)),
            pl.BlockSpec((L, MBLK), lambda i: (0, i)),
        ],
        out_specs=[
            pl.BlockSpec((MBLK, L), lambda i: (i, 0)),
            pl.BlockSpec((MBLK, L), lambda i: (i, 0)),
        ],
        out_shape=[
            jax.ShapeDtypeStruct((B, L), jnp.float32),
            jax.ShapeDtypeStruct((B, L), jnp.float32),
        ],
    )(gathered, p_t)

    return (effect, prioritization[..., None])


# speedup vs baseline: 1.2542x; 1.2542x over previous
import functools

import jax
import jax.numpy as jnp
from jax import lax
from jax.experimental import pallas as pl
from jax.experimental.pallas import tpu as pltpu
from jax.experimental.pallas import tpu_sc as plsc

B, L, D = 16384, 50, 64
BL = B * L
N_TABLE = 1000000

CHUNK = 128
_NW = 32
N_PER_W = BL // _NW
N_CHUNKS = N_PER_W // CHUNK
K_INFLIGHT = 8
N_GROUPS = N_CHUNKS // K_INFLIGHT


def _make_sc_gather():
    info = plsc.get_sparse_core_info()
    nc, ns = info.num_cores, info.num_subcores
    assert nc * ns == _NW
    mesh = plsc.VectorSubcoreMesh(core_axis_name="c", subcore_axis_name="s")

    @functools.partial(
        pl.kernel,
        mesh=mesh,
        out_type=jax.ShapeDtypeStruct((_NW, N_CHUNKS, CHUNK), jnp.float32),
        scratch_types=[
            pltpu.VMEM((N_CHUNKS, CHUNK), jnp.int32),
            pltpu.VMEM((N_CHUNKS, CHUNK), jnp.float32),
            pltpu.VMEM_SHARED((N_TABLE,), jnp.float32),
            pltpu.SemaphoreType.DMA,
        ],
    )
    def gather_k(table_hbm, idx_hbm, out_hbm, idx_v, rows_v, table_s, sem):
        sid = lax.axis_index("s")
        wid = sid * nc + lax.axis_index("c")

        @pl.when(sid == 0)
        def _():
            pltpu.sync_copy(table_hbm, table_s)

        pltpu.sync_copy(idx_hbm.at[wid], idx_v)
        plsc.subcore_barrier()

        def group(g, _):
            base = g * K_INFLIGHT
            descs = []
            for k in range(K_INFLIGHT):
                descs.append(
                    pltpu.async_copy(
                        table_s.at[idx_v.at[base + k]], rows_v.at[base + k], sem
                    )
                )
            for d in descs:
                d.wait()
            return 0

        lax.fori_loop(0, N_GROUPS, group, 0)
        pltpu.sync_copy(rows_v, out_hbm.at[wid])

    return gather_k


_sc_gather = _make_sc_gather()


BBLK = 1024


def _tc_body(x_ref, w_ref, bvec_ref, bias_ref, p_ref):
    x = x_ref[...].reshape(L, D, BBLK)
    w3 = w_ref[...].reshape(1, D, 1)
    s = jnp.sum(x * w3, axis=1)
    c = jnp.sum(bias_ref[...] * w_ref[...]) + bvec_ref[0, 0]
    p_ref[...] = jax.nn.sigmoid(s + c)


MBLK = 4096


def _fin_body(g_ref, p_ref, eff_ref):
    eff_ref[...] = g_ref[...] * p_ref[...]


def kernel(variantxgene_embedding, variantxgene_ixs, W, b, variantxgene_effect, embedding_bias):
    idx_t = (
        variantxgene_ixs.astype(jnp.int32)
        .T.reshape(_NW, N_CHUNKS, CHUNK)
    )
    gathered = _sc_gather(variantxgene_effect, idx_t).reshape(L, B)

    x2 = variantxgene_embedding.transpose(1, 2, 0).reshape(L * D, B)

    w_col = W.reshape(D, 1)
    bias_col = embedding_bias.reshape(D, 1)
    b2 = b.reshape(1, 1)

    grid = (B // BBLK,)
    p_t = pl.pallas_call(
        _tc_body,
        grid=grid,
        in_specs=[
            pl.BlockSpec((L * D, BBLK), lambda i: (0, i)),
            pl.BlockSpec((D, 1), lambda i: (0, 0)),
            pl.BlockSpec((1, 1), lambda i: (0, 0)),
            pl.BlockSpec((D, 1), lambda i: (0, 0)),
        ],
        out_specs=pl.BlockSpec((L, BBLK), lambda i: (0, i)),
        out_shape=jax.ShapeDtypeStruct((L, B), jnp.float32),
    )(x2, w_col, b2, bias_col)

    eff_t = pl.pallas_call(
        _fin_body,
        grid=(B // MBLK,),
        in_specs=[
            pl.BlockSpec((L, MBLK), lambda i: (0, i)),
            pl.BlockSpec((L, MBLK), lambda i: (0, i)),
        ],
        out_specs=pl.BlockSpec((L, MBLK), lambda i: (0, i)),
        out_shape=jax.ShapeDtypeStruct((L, B), jnp.float32),
    )(gathered, p_t)

    effect = eff_t.T
    prioritization = p_t.T[..., None]
    return (effect, prioritization)
